# baseline (speedup 1.0000x reference)
import jax
import jax.numpy as jnp
from jax.experimental import pallas as pl

N_NODES_K = 10000
NUM_GRAPHS_K = 64


def _gcn_conv(x, src, dst, W, b, num_nodes):
    loops = jnp.arange(num_nodes, dtype=src.dtype)
    src = jnp.concatenate([src, loops])
    dst = jnp.concatenate([dst, loops])
    deg = jnp.zeros((num_nodes,), dtype=x.dtype).at[dst].add(1.0)
    dinv_sqrt = jnp.where(deg > 0, 1.0 / jnp.sqrt(deg), 0.0)
    norm = dinv_sqrt[src] * dinv_sqrt[dst]
    xw = x @ W
    msg = xw[src] * norm[:, None]
    out = jnp.zeros((num_nodes, W.shape[1]), dtype=x.dtype).at[dst].add(msg)
    return out + b


def _head_kernel(pooled_ref, wo_ref, bo_ref, out_ref):
    out_ref[...] = (
        jnp.dot(pooled_ref[...], wo_ref[...], preferred_element_type=jnp.float32)
        + bo_ref[...]
    )


def kernel(x, edge_index, batch, W1, b1, W2, b2, Wo, bo):
    src = edge_index[0].astype(jnp.int32)
    dst = edge_index[1].astype(jnp.int32)
    h = jax.nn.relu(_gcn_conv(x, src, dst, W1, b1, N_NODES_K))
    h = jax.nn.relu(_gcn_conv(h, src, dst, W2, b2, N_NODES_K))
    sums = jax.ops.segment_sum(h, batch, num_segments=NUM_GRAPHS_K)
    counts = jax.ops.segment_sum(jnp.ones((N_NODES_K,), h.dtype), batch,
                                 num_segments=NUM_GRAPHS_K)
    pooled = sums / jnp.maximum(counts, 1.0)[:, None]
    out = pl.pallas_call(
        _head_kernel,
        out_shape=jax.ShapeDtypeStruct((NUM_GRAPHS_K, Wo.shape[1]), jnp.float32),
    )(pooled, Wo, bo[None, :])
    return out


# speedup vs baseline: 8.0728x; 8.0728x over previous
import functools

import jax
import jax.numpy as jnp
from jax import lax
from jax.experimental import pallas as pl
from jax.experimental.pallas import tpu as pltpu
from jax.experimental.pallas import tpu_sc as plsc

N = 10000
D = 128
G = 64
DO = 512
NC, NS, L = 2, 16, 16
NW = NC * NS
K = 128
NP = 10240
ROWS_PT = NP // NS
DEGW = 128
BM = 512

def _sc_mesh():
    return plsc.VectorSubcoreMesh(
        core_axis_name="c", subcore_axis_name="s", num_cores=NC, num_subcores=NS)


def _num_chunks(num_edges):
    return -(-num_edges // (NW * K))


NPR = NP // 128
DROWS_PT = 8


def _deg_body(cpt, dst_hbm, eye_hbm, degp_hbm, didx_v, ridx_v, cidx_v,
              rows_v, zrow_v, deg_sp, sem):
    c = lax.axis_index("c")
    s = lax.axis_index("s")
    wid = c * NS + s
    base = wid * (cpt * K)

    def fill_zero(r, _):
        for j in range(DEGW // L):
            zrow_v[r, pl.ds(j * L, L)] = jnp.zeros((L,), jnp.float32)
        return 0
    lax.fori_loop(0, DROWS_PT, fill_zero, 0)

    @pl.when(s < NPR // DROWS_PT)
    def _():
        pltpu.sync_copy(zrow_v, deg_sp.at[pl.ds(s * DROWS_PT, DROWS_PT)])
    plsc.subcore_barrier()

    def chunk(i, _):
        pltpu.sync_copy(dst_hbm.at[pl.ds(base + i * K, K)], didx_v)
        for j in range(K // L):
            v = didx_v[pl.ds(j * L, L)]
            ridx_v[pl.ds(j * L, L)] = lax.shift_right_logical(v, 7)
            cidx_v[pl.ds(j * L, L)] = lax.bitwise_and(v, 127)
        pltpu.async_copy(eye_hbm.at[cidx_v], rows_v, sem).wait()
        pltpu.sync_copy(rows_v, deg_sp.at[ridx_v], add=True)
        return 0
    lax.fori_loop(0, cpt, chunk, 0)
    plsc.subcore_barrier()

    @pl.when(s < NPR // DROWS_PT)
    def _():
        pltpu.sync_copy(deg_sp.at[pl.ds(s * DROWS_PT, DROWS_PT)], zrow_v)
        pltpu.sync_copy(zrow_v, degp_hbm.at[c, pl.ds(s * DROWS_PT, DROWS_PT)])


def _make_deg_call(cpt):
    return functools.partial(
        pl.kernel,
        out_type=jax.ShapeDtypeStruct((NC, NPR, DEGW), jnp.float32),
        mesh=_sc_mesh(),
        scratch_types=[
            pltpu.VMEM((K,), jnp.int32),
            pltpu.VMEM((K,), jnp.int32),
            pltpu.VMEM((K,), jnp.int32),
            pltpu.VMEM((K, DEGW), jnp.float32),
            pltpu.VMEM((DROWS_PT, DEGW), jnp.float32),
            pltpu.VMEM_SHARED((NPR, DEGW), jnp.float32),
            pltpu.SemaphoreType.DMA,
        ],
    )(functools.partial(_deg_body, cpt))


def _msg_body(cpt, src_hbm, dst_hbm, y_hbm, accp_hbm,
              sidx_v, didx_v, rows_v, acc_sp, sem):
    c = lax.axis_index("c")
    s = lax.axis_index("s")
    wid = c * NS + s
    base = wid * (cpt * K)

    def zrow(r, _):
        for j in range(D // L):
            rows_v[r, pl.ds(j * L, L)] = jnp.zeros((L,), jnp.float32)
        return 0
    lax.fori_loop(0, K, zrow, 0)
    for j in range(ROWS_PT // K):
        pltpu.sync_copy(rows_v, acc_sp.at[pl.ds(s * ROWS_PT + j * K, K)])
    plsc.subcore_barrier()

    def chunk(i, _):
        pltpu.sync_copy(src_hbm.at[pl.ds(base + i * K, K)], sidx_v)
        pltpu.sync_copy(dst_hbm.at[pl.ds(base + i * K, K)], didx_v)
        pltpu.async_copy(y_hbm.at[sidx_v], rows_v, sem).wait()
        pltpu.sync_copy(rows_v, acc_sp.at[didx_v], add=True)
        return 0
    lax.fori_loop(0, cpt, chunk, 0)
    plsc.subcore_barrier()
    for j in range(ROWS_PT // K):
        r0 = s * ROWS_PT + j * K
        pltpu.sync_copy(acc_sp.at[pl.ds(r0, K)], rows_v)
        pltpu.sync_copy(rows_v, accp_hbm.at[c, pl.ds(r0, K)])


def _make_msg_call(cpt):
    return functools.partial(
        pl.kernel,
        out_type=jax.ShapeDtypeStruct((NC, NP, D), jnp.float32),
        mesh=_sc_mesh(),
        scratch_types=[
            pltpu.VMEM((K,), jnp.int32),
            pltpu.VMEM((K,), jnp.int32),
            pltpu.VMEM((K, D), jnp.float32),
            pltpu.VMEM_SHARED((NP, D), jnp.float32),
            pltpu.SemaphoreType.DMA,
        ],
    )(functools.partial(_msg_body, cpt))


def _dinv_of(degp_ref):
    d = degp_ref[0] + degp_ref[1] + 1.0
    return lax.rsqrt(d)


def _scale_body(x_ref, w_ref, degp_ref, y_ref):
    dinv = _dinv_of(degp_ref)
    y_ref[...] = dinv * jnp.dot(x_ref[...], w_ref[...],
                                preferred_element_type=jnp.float32)


def _layer2_body(accp_ref, y1_ref, degp_ref, b1_ref, w2_ref, y2_ref):
    dinv = _dinv_of(degp_ref)
    h = jnp.maximum(
        dinv * (accp_ref[0] + accp_ref[1] + y1_ref[...]) + b1_ref[...], 0.0)
    y2_ref[...] = dinv * jnp.dot(h, w2_ref[...],
                                 preferred_element_type=jnp.float32)


def _final_body(nb, accp_ref, y2_ref, degp_ref, b2_ref, batch_ref, wo_ref,
                bo_ref, out_ref, sums, cnts):
    m = pl.program_id(0)

    @pl.when(m == 0)
    def _():
        sums[...] = jnp.zeros_like(sums)
        cnts[...] = jnp.zeros_like(cnts)

    dinv = _dinv_of(degp_ref)
    h = jnp.maximum(
        dinv * (accp_ref[0] + accp_ref[1] + y2_ref[...]) + b2_ref[...], 0.0)
    oh = (batch_ref[...] == lax.broadcasted_iota(jnp.int32, (G, BM), 0)
          ).astype(jnp.float32)
    sums[...] += jnp.dot(oh, h, preferred_element_type=jnp.float32)
    cnts[...] = cnts[...] + jnp.sum(oh, axis=1, keepdims=True)

    @pl.when(m == nb - 1)
    def _():
        pooled = sums[...] / jnp.maximum(cnts[...], 1.0)
        out_ref[...] = jnp.dot(pooled, wo_ref[...],
                               preferred_element_type=jnp.float32) + bo_ref[...]


_NB = NP // BM

_scale_call = pl.pallas_call(
    _scale_body,
    grid=(_NB,),
    in_specs=[
        pl.BlockSpec((BM, D), lambda i: (i, 0)),
        pl.BlockSpec((D, D), lambda i: (0, 0)),
        pl.BlockSpec((NC, BM, 1), lambda i: (0, i, 0)),
    ],
    out_specs=pl.BlockSpec((BM, D), lambda i: (i, 0)),
    out_shape=jax.ShapeDtypeStruct((NP, D), jnp.float32),
)

_layer2_call = pl.pallas_call(
    _layer2_body,
    grid=(_NB,),
    in_specs=[
        pl.BlockSpec((NC, BM, D), lambda i: (0, i, 0)),
        pl.BlockSpec((BM, D), lambda i: (i, 0)),
        pl.BlockSpec((NC, BM, 1), lambda i: (0, i, 0)),
        pl.BlockSpec((1, D), lambda i: (0, 0)),
        pl.BlockSpec((D, D), lambda i: (0, 0)),
    ],
    out_specs=pl.BlockSpec((BM, D), lambda i: (i, 0)),
    out_shape=jax.ShapeDtypeStruct((NP, D), jnp.float32),
)

_final_call = pl.pallas_call(
    functools.partial(_final_body, _NB),
    grid=(_NB,),
    in_specs=[
        pl.BlockSpec((NC, BM, D), lambda i: (0, i, 0)),
        pl.BlockSpec((BM, D), lambda i: (i, 0)),
        pl.BlockSpec((NC, BM, 1), lambda i: (0, i, 0)),
        pl.BlockSpec((1, D), lambda i: (0, 0)),
        pl.BlockSpec((1, BM), lambda i: (0, i)),
        pl.BlockSpec((D, DO), lambda i: (0, 0)),
        pl.BlockSpec((1, DO), lambda i: (0, 0)),
    ],
    out_specs=pl.BlockSpec((G, DO), lambda i: (0, 0)),
    out_shape=jax.ShapeDtypeStruct((G, DO), jnp.float32),
    scratch_shapes=[
        pltpu.VMEM((G, D), jnp.float32),
        pltpu.VMEM((G, D), jnp.float32),
    ],
)


def kernel(x, edge_index, batch, W1, b1, W2, b2, Wo, bo):
    num_edges = edge_index.shape[1]
    cpt = _num_chunks(num_edges)
    ep = NW * cpt * K
    src = edge_index[0].astype(jnp.int32)
    dst = edge_index[1].astype(jnp.int32)
    pad = jnp.full((ep - num_edges,), N, jnp.int32)
    src_p = jnp.concatenate([src, pad])
    dst_p = jnp.concatenate([dst, pad])
    n = x.shape[0]
    x_p = jnp.concatenate([x, jnp.zeros((NP - n, D), jnp.float32)])
    batch_p = jnp.concatenate(
        [batch.astype(jnp.int32), jnp.full((NP - n,), G, jnp.int32)])[None, :]

    deg_call = _make_deg_call(cpt)
    msg_call = _make_msg_call(cpt)

    eye = jnp.eye(DEGW, dtype=jnp.float32)
    degp = deg_call(dst_p, eye).reshape(NC, NP, 1)
    y1 = _scale_call(x_p, W1, degp)
    acc1 = msg_call(src_p, dst_p, y1)
    y2 = _layer2_call(acc1, y1, degp, b1[None, :], W2)
    acc2 = msg_call(src_p, dst_p, y2)
    out = _final_call(acc2, y2, degp, b2[None, :], batch_p, Wo, bo[None, :])
    return out
